# baseline (speedup 1.0000x reference)
import functools

import jax
import jax.numpy as jnp
from jax import lax
from jax.experimental import pallas as pl
from jax.experimental.pallas import tpu as pltpu
from jax.experimental.pallas import tpu_sc as plsc

N = 10000
E = 320000
D = 128
ALPHA = 0.1
BETA = 0.5

NC = 2
NS = 16
NW = NC * NS
EPW = E // NW
EB = 80
NB = EPW // EB
DEPTH = 2
IB = 8
NPAD = 10240
RPT = NPAD // NS


def _sc_body(feat_h, src_h, dst_h, ew_h, out_h,
             rows_v, srows_v, ibi_v, ibd_v, ibw_v, wbb_v, acc_sh,
             g0, g1, s0, s1,
             i0, i1, i2, i3, i4, i5, i6, i7):
    gsem = (g0, g1)
    ssem = (s0, s1)
    isem = (i0, i1, i2, i3, i4, i5, i6, i7)
    c = lax.axis_index("c")
    s = lax.axis_index("s")
    wid = s * NC + c
    r0 = s * RPT
    ebase = wid * EPW

    def start_fetch(b, sl):
        off = ebase + b * EB
        pltpu.async_copy(src_h.at[pl.ds(off, EB)], ibi_v.at[sl], isem[sl])
        pltpu.async_copy(dst_h.at[pl.ds(off, EB)], ibd_v.at[sl], isem[sl])
        pltpu.async_copy(ew_h.at[pl.ds(off, EB)], ibw_v.at[sl], isem[sl])

    def wait_fetch(sl):
        pltpu.make_async_copy(src_h.at[pl.ds(0, EB)], ibi_v.at[sl],
                              isem[sl]).wait()
        pltpu.make_async_copy(dst_h.at[pl.ds(0, EB)], ibd_v.at[sl],
                              isem[sl]).wait()
        pltpu.make_async_copy(ew_h.at[pl.ds(0, EB)], ibw_v.at[sl],
                              isem[sl]).wait()

    def start_gather(t, it):
        pltpu.async_copy(feat_h.at[ibi_v.at[it]], rows_v.at[t], gsem[t])

    def wait_gather(t, it):
        pltpu.make_async_copy(feat_h.at[ibi_v.at[it]], rows_v.at[t],
                              gsem[t]).wait()

    def start_scatter(t, it):
        pltpu.async_copy(srows_v.at[t], acc_sh.at[ibd_v.at[it]], ssem[t],
                         add=True)

    def wait_scatter(t):
        pltpu.make_async_copy(srows_v.at[t], acc_sh.at[ibd_v.at[t % IB]],
                              ssem[t]).wait()

    def scale(t, it):
        @pl.loop(0, EB // 16)
        def _g(g):
            wv = ibw_v[it, pl.ds(g * 16, 16)]
            for k in range(16):
                wbb_v[pl.ds((g * 16 + k) * 16, 16)] = jnp.full(
                    (16,), wv[k], jnp.float32)

        @pl.loop(0, EB)
        def _e(e):
            wb = wbb_v[pl.ds(e * 16, 16)]
            for ch in range(D // 16):
                sl = pl.ds(ch * 16, 16)
                srows_v[t, e, sl] = rows_v[t, e, sl] * wb

    def step(b, j):
        t = j % DEPTH
        tg = (j + 1) % DEPTH
        it = j
        itg = (j + 1) % IB
        itf = (j + 5) % IB
        static = isinstance(b, int)
        if (not static) or (b + 5 < NB):
            start_fetch(b + 5, itf)
        if (not static) or (b + 1 < NB):
            wait_fetch(itg)
            start_gather(tg, itg)
        wait_gather(t, it)
        if (not static) or (b >= DEPTH):
            wait_scatter(t)
        scale(t, it)
        start_scatter(t, it)

    for bp in range(5):
        start_fetch(bp, bp)
    zvec = jnp.zeros((16,), jnp.float32)

    @pl.loop(0, EB)
    def _z(e):
        for ch in range(D // 16):
            srows_v[0, e, pl.ds(ch * 16, 16)] = zvec

    zsem = gsem + ssem
    for r in range(RPT // EB // 4):
        for k4 in range(4):
            k = r * 4 + k4
            pltpu.async_copy(srows_v.at[0],
                             acc_sh.at[pl.ds(r0 + k * EB, EB)], zsem[k4])
        for k4 in range(4):
            k = r * 4 + k4
            pltpu.make_async_copy(srows_v.at[0],
                                  acc_sh.at[pl.ds(r0 + k * EB, EB)],
                                  zsem[k4]).wait()
    wait_fetch(0)
    start_gather(0, 0)
    plsc.subcore_barrier()

    for b in range(8):
        step(b, b)

    @pl.loop(0, (NB - 8 - (NB % 8)) // 8)
    def _i(i):
        for j in range(8):
            step(8 + i * 8 + j, j)

    for b in range(NB - (NB % 8) - (8 if NB % 8 == 0 else 0), NB):
        step(b, b % 8)

    for t in range(DEPTH):
        wait_scatter(t)

    plsc.subcore_barrier()
    pltpu.sync_copy(acc_sh.at[pl.ds(r0, RPT)], out_h.at[c, pl.ds(r0, RPT)])


@jax.jit
def _sc_propagate(features, src, dst, edge_weight):
    mesh = plsc.VectorSubcoreMesh(core_axis_name="c", subcore_axis_name="s")
    f = pl.kernel(
        _sc_body,
        out_type=jax.ShapeDtypeStruct((NC, NPAD, D), jnp.float32),
        mesh=mesh,
        scratch_types=[
            pltpu.VMEM((DEPTH, EB, D), jnp.float32),
            pltpu.VMEM((DEPTH, EB, D), jnp.float32),
            pltpu.VMEM((IB, EB), jnp.int32),
            pltpu.VMEM((IB, EB), jnp.int32),
            pltpu.VMEM((IB, EB), jnp.float32),
            pltpu.VMEM((EB * 16,), jnp.float32),
            pltpu.VMEM_SHARED((NPAD, D), jnp.float32),
        ] + [pltpu.SemaphoreType.DMA] * (2 * DEPTH + IB),
    )
    return f(features, src, dst, edge_weight)


def _tc_body(p_ref, f0_ref, w_ref, o_ref):
    hi = p_ref[0] + p_ref[1]
    support = (1.0 - ALPHA) * hi + ALPHA * f0_ref[...]
    out = BETA * jnp.dot(support, w_ref[...],
                         preferred_element_type=jnp.float32)
    out = out + (1.0 - BETA) * support
    o_ref[...] = jnp.maximum(out, 0.0)


@jax.jit
def _tc_combine(partials, features0, W):
    RB = 2000
    return pl.pallas_call(
        _tc_body,
        grid=(N // RB,),
        in_specs=[
            pl.BlockSpec((NC, RB, D), lambda i: (0, i, 0)),
            pl.BlockSpec((RB, D), lambda i: (i, 0)),
            pl.BlockSpec((D, D), lambda i: (0, 0)),
        ],
        out_specs=pl.BlockSpec((RB, D), lambda i: (i, 0)),
        out_shape=jax.ShapeDtypeStruct((N, D), jnp.float32),
    )(partials, features0, W)


def kernel(features, features0, edge_index, edge_weight, W):
    partials = _sc_propagate(features, edge_index[0], edge_index[1], edge_weight)
    return _tc_combine(partials, features0, W)


# speedup vs baseline: 1.0222x; 1.0222x over previous
import functools

import jax
import jax.numpy as jnp
from jax import lax
from jax.experimental import pallas as pl
from jax.experimental.pallas import tpu as pltpu
from jax.experimental.pallas import tpu_sc as plsc

N = 10000
E = 320000
D = 128
ALPHA = 0.1
BETA = 0.5

NC = 2
NS = 16
NW = NC * NS
EPW = E // NW
EB = 80
NB = EPW // EB
DEPTH = 4
IB = 8
NPAD = 10240
RPT = NPAD // NS


def _sc_body(feat_h, src_h, dst_h, ew_h, out_h,
             rows_v, ibi_v, ibd_v, ibw_v, acc_sh,
             g0, g1, g2, g3, s0, s1, s2, s3,
             i0, i1, i2, i3, i4, i5, i6, i7):
    gsem = (g0, g1, g2, g3)
    ssem = (s0, s1, s2, s3)
    isem = (i0, i1, i2, i3, i4, i5, i6, i7)
    c = lax.axis_index("c")
    s = lax.axis_index("s")
    wid = s * NC + c
    r0 = s * RPT
    ebase = wid * EPW

    def start_fetch(b, sl):
        off = ebase + b * EB
        pltpu.async_copy(src_h.at[pl.ds(off, EB)], ibi_v.at[sl], isem[sl])
        pltpu.async_copy(dst_h.at[pl.ds(off, EB)], ibd_v.at[sl], isem[sl])
        pltpu.async_copy(ew_h.at[pl.ds(off, EB)], ibw_v.at[sl], isem[sl])

    def wait_fetch(sl):
        pltpu.make_async_copy(src_h.at[pl.ds(0, EB)], ibi_v.at[sl],
                              isem[sl]).wait()
        pltpu.make_async_copy(dst_h.at[pl.ds(0, EB)], ibd_v.at[sl],
                              isem[sl]).wait()
        pltpu.make_async_copy(ew_h.at[pl.ds(0, EB)], ibw_v.at[sl],
                              isem[sl]).wait()

    def start_gather(t, it):
        pltpu.async_copy(feat_h.at[ibi_v.at[it]], rows_v.at[t], gsem[t])

    def wait_gather(t, it):
        pltpu.make_async_copy(feat_h.at[ibi_v.at[it]], rows_v.at[t],
                              gsem[t]).wait()

    def start_scatter(t, it):
        pltpu.async_copy(rows_v.at[t], acc_sh.at[ibd_v.at[it]], ssem[t],
                         add=True)

    def wait_scatter(t):
        pltpu.make_async_copy(rows_v.at[t], acc_sh.at[ibd_v.at[t % IB]],
                              ssem[t]).wait()

    def scale(t, it):
        @pl.loop(0, EB // 16)
        def _g(g):
            wv = ibw_v[it, pl.ds(g * 16, 16)]
            for k in range(16):
                w = wv[k]
                e = g * 16 + k
                for ch in range(D // 16):
                    sl = pl.ds(ch * 16, 16)
                    rows_v[t, e, sl] = rows_v[t, e, sl] * w

    def step(b, j):
        t = j % DEPTH
        tg = (j + 3) % DEPTH
        it = j
        itg = (j + 3) % IB
        itf = (j + 5) % IB
        static = isinstance(b, int)
        if (not static) or (b + 5 < NB):
            start_fetch(b + 5, itf)
        if (not static) or (b + 3 < NB):
            if (not static) or (b >= 1):
                wait_scatter(tg)
            wait_fetch(itg)
            start_gather(tg, itg)
        wait_gather(t, it)
        scale(t, it)
        start_scatter(t, it)

    for bp in range(5):
        start_fetch(bp, bp)
    zvec = jnp.zeros((16,), jnp.float32)

    @pl.loop(0, EB)
    def _z(e):
        for ch in range(D // 16):
            rows_v[0, e, pl.ds(ch * 16, 16)] = zvec

    for k in range(RPT // EB):
        pltpu.sync_copy(rows_v.at[0], acc_sh.at[pl.ds(r0 + k * EB, EB)])
    for bp in range(3):
        wait_fetch(bp)
        start_gather(bp, bp)
    plsc.subcore_barrier()

    for b in range(8):
        step(b, b)

    @pl.loop(0, (NB - 8 - (NB % 8)) // 8)
    def _i(i):
        for j in range(8):
            step(8 + i * 8 + j, j)

    for b in range(NB - (NB % 8) - (8 if NB % 8 == 0 else 0), NB):
        step(b, b % 8)

    for t in range(DEPTH):
        wait_scatter(t)

    plsc.subcore_barrier()
    pltpu.sync_copy(acc_sh.at[pl.ds(r0, RPT)], out_h.at[c, pl.ds(r0, RPT)])


@jax.jit
def _sc_propagate(features, src, dst, edge_weight):
    mesh = plsc.VectorSubcoreMesh(core_axis_name="c", subcore_axis_name="s")
    f = pl.kernel(
        _sc_body,
        out_type=jax.ShapeDtypeStruct((NC, NPAD, D), jnp.float32),
        mesh=mesh,
        scratch_types=[
            pltpu.VMEM((DEPTH, EB, D), jnp.float32),
            pltpu.VMEM((IB, EB), jnp.int32),
            pltpu.VMEM((IB, EB), jnp.int32),
            pltpu.VMEM((IB, EB), jnp.float32),
            pltpu.VMEM_SHARED((NPAD, D), jnp.float32),
        ] + [pltpu.SemaphoreType.DMA] * (2 * DEPTH + IB),
    )
    return f(features, src, dst, edge_weight)


def _tc_body(p_ref, f0_ref, w_ref, o_ref):
    hi = p_ref[0] + p_ref[1]
    support = (1.0 - ALPHA) * hi + ALPHA * f0_ref[...]
    out = BETA * jnp.dot(support, w_ref[...],
                         preferred_element_type=jnp.float32)
    out = out + (1.0 - BETA) * support
    o_ref[...] = jnp.maximum(out, 0.0)


@jax.jit
def _tc_combine(partials, features0, W):
    RB = 2000
    return pl.pallas_call(
        _tc_body,
        grid=(N // RB,),
        in_specs=[
            pl.BlockSpec((NC, RB, D), lambda i: (0, i, 0)),
            pl.BlockSpec((RB, D), lambda i: (i, 0)),
            pl.BlockSpec((D, D), lambda i: (0, 0)),
        ],
        out_specs=pl.BlockSpec((RB, D), lambda i: (i, 0)),
        out_shape=jax.ShapeDtypeStruct((N, D), jnp.float32),
    )(partials, features0, W)


def kernel(features, features0, edge_index, edge_weight, W):
    partials = _sc_propagate(features, edge_index[0], edge_index[1], edge_weight)
    return _tc_combine(partials, features0, W)
